# Initial kernel scaffold; baseline (speedup 1.0000x reference)
import jax
import jax.numpy as jnp
from jax.experimental import pallas as pl


def kernel(R, neighbor_idx):
    raise NotImplementedError("write your pallas kernel here")


# speedup vs baseline: 32.1696x; 32.1696x over previous
import functools

import jax
import jax.numpy as jnp
from jax import lax
from jax.experimental import pallas as pl
from jax.experimental.pallas import tpu as pltpu
from jax.experimental.pallas import tpu_sc as plsc

_N = 100000
_E = 6400000
_NB = 5
_RMIN, _RMAX = 0.5, 6.0
_BETA = _NB**2 / _RMAX**2
_SHIFTS = tuple(_RMIN + (_RMAX - _RMIN) / _NB * k for k in range(_NB))

_NC, _NS = 2, 16
_NW = _NC * _NS
_EPW = _E // _NW
_C = 2000
_NCHUNK = _EPW // _C
_GPC = _C // 16
_NP = 100352
_RPT = _NP // _NS

_mesh = plsc.VectorSubcoreMesh(
    core_axis_name="c", subcore_axis_name="s", num_cores=_NC, num_subcores=_NS
)


@functools.partial(
    pl.kernel,
    mesh=_mesh,
    out_type=jax.ShapeDtypeStruct((_NC, _NB, _NP), jnp.float32),
    compiler_params=pltpu.CompilerParams(use_tc_tiling_on_sc=False),
    scratch_types=[
        [pltpu.VMEM_SHARED((_NP,), jnp.float32) for _ in range(_NB)],
        pltpu.VMEM((_C,), jnp.int32),
        pltpu.VMEM((_C,), jnp.int32),
        [pltpu.VMEM((_C,), jnp.float32) for _ in range(6)],
        [pltpu.VMEM((_C,), jnp.float32) for _ in range(_NB)],
        pltpu.SemaphoreType.DMA,
    ],
)
def _rbf_sc(xh, yh, zh, i0h, i1h, out, acc, i0, i1, comp, ebufs, gsem):
    cid = lax.axis_index("c")
    sid = lax.axis_index("s")
    wid = cid * _NS + sid
    zero16 = jnp.zeros((16,), jnp.float32)

    def _zb(i, _):
        ebufs[0][pl.ds(i * 16, 16)] = zero16
        return 0

    lax.fori_loop(0, _C // 16, _zb, 0)
    r0 = sid * _RPT
    for k in range(_NB):
        for t in range(_RPT // _C):
            pltpu.sync_copy(ebufs[0], acc[k].at[pl.ds(r0 + t * _C, _C)])
        rem = _RPT % _C
        if rem:
            pltpu.sync_copy(
                ebufs[0].at[pl.ds(0, rem)],
                acc[k].at[pl.ds(r0 + _RPT - rem, rem)],
            )
    plsc.subcore_barrier()

    def _chunk(ch, _):
        base = wid * _EPW + ch * _C
        pltpu.sync_copy(i0h.at[pl.ds(base, _C)], i0)
        pltpu.sync_copy(i1h.at[pl.ds(base, _C)], i1)
        g = [
            pltpu.async_copy(xh.at[i0], comp[0], gsem),
            pltpu.async_copy(yh.at[i0], comp[1], gsem),
            pltpu.async_copy(zh.at[i0], comp[2], gsem),
            pltpu.async_copy(xh.at[i1], comp[3], gsem),
            pltpu.async_copy(yh.at[i1], comp[4], gsem),
            pltpu.async_copy(zh.at[i1], comp[5], gsem),
        ]
        for d in g:
            d.wait()

        def _group(gi, _):
            sl = pl.ds(gi * 16, 16)
            dx = comp[0][sl] - comp[3][sl]
            dy = comp[1][sl] - comp[4][sl]
            dz = comp[2][sl] - comp[5][sl]
            d2 = dx * dx + dy * dy + dz * dz
            bits = lax.bitcast_convert_type(d2, jnp.int32)
            y = lax.bitcast_convert_type(
                jnp.int32(0x5F3759DF) - (bits >> 1), jnp.float32
            )
            for _i in range(3):
                y = y * (1.5 - 0.5 * d2 * y * y)
            dr = d2 * y
            for k in range(_NB):
                t = _SHIFTS[k] - dr
                ebufs[k][sl] = jnp.exp((-_BETA) * (t * t))
            return 0

        lax.fori_loop(0, _GPC, _group, 0)
        for k in range(_NB):
            pltpu.sync_copy(ebufs[k], acc[k].at[i1], add=True)
        return 0

    lax.fori_loop(0, _NCHUNK, _chunk, 0)

    plsc.subcore_barrier()
    for k in range(_NB):
        pltpu.sync_copy(
            acc[k].at[pl.ds(r0, _RPT)], out.at[cid, k, pl.ds(r0, _RPT)]
        )


def _add_body(p_ref, o_ref):
    o_ref[...] = p_ref[0] + p_ref[1]


def kernel(R, neighbor_idx):
    R = R.astype(jnp.float32)
    x, y, z = R[:, 0], R[:, 1], R[:, 2]
    i0 = neighbor_idx[0].astype(jnp.int32)
    i1 = neighbor_idx[1].astype(jnp.int32)
    partial = _rbf_sc(x, y, z, i0, i1)
    p2 = partial.reshape(_NC, _NB * _NP // 128, 128)
    summed = pl.pallas_call(
        _add_body,
        out_shape=jax.ShapeDtypeStruct((_NB * _NP // 128, 128), jnp.float32),
    )(p2)
    return summed.reshape(_NB, _NP)[:, :_N].T


# speedup vs baseline: 40.1125x; 1.2469x over previous
import functools

import jax
import jax.numpy as jnp
from jax import lax
from jax.experimental import pallas as pl
from jax.experimental.pallas import tpu as pltpu
from jax.experimental.pallas import tpu_sc as plsc

_N = 100000
_E = 6400000
_NB = 5
_RMIN, _RMAX = 0.5, 6.0
_BETA = _NB**2 / _RMAX**2
_SHIFTS = tuple(_RMIN + (_RMAX - _RMIN) / _NB * k for k in range(_NB))

_NC, _NS = 2, 16
_NW = _NC * _NS
_EPW = _E // _NW
_C = 2000
_NCHUNK = _EPW // _C
_GPC = _C // 16
_NP = 100352
_RPT = _NP // _NS

_mesh = plsc.VectorSubcoreMesh(
    core_axis_name="c", subcore_axis_name="s", num_cores=_NC, num_subcores=_NS
)


@functools.partial(
    pl.kernel,
    mesh=_mesh,
    out_type=jax.ShapeDtypeStruct((_NC, _NB, _NP), jnp.float32),
    compiler_params=pltpu.CompilerParams(use_tc_tiling_on_sc=False),
    scratch_types=[
        [pltpu.VMEM_SHARED((_NP,), jnp.float32) for _ in range(_NB)],
        [pltpu.VMEM((_C,), jnp.int32) for _ in range(2)],
        [pltpu.VMEM((_C,), jnp.int32) for _ in range(2)],
        [[pltpu.VMEM((_C,), jnp.float32) for _ in range(6)] for _ in range(2)],
        [pltpu.VMEM((_C,), jnp.float32) for _ in range(_NB)],
        [pltpu.SemaphoreType.DMA for _ in range(2)],
    ],
)
def _rbf_sc(xh, yh, zh, i0h, i1h, out, acc, i0b, i1b, comp, ebufs, gsems):
    cid = lax.axis_index("c")
    sid = lax.axis_index("s")
    wid = cid * _NS + sid
    zero16 = jnp.zeros((16,), jnp.float32)

    def _zb(i, _):
        ebufs[0][pl.ds(i * 16, 16)] = zero16
        return 0

    lax.fori_loop(0, _C // 16, _zb, 0)
    r0 = sid * _RPT
    for k in range(_NB):
        for t in range(_RPT // _C):
            pltpu.sync_copy(ebufs[0], acc[k].at[pl.ds(r0 + t * _C, _C)])
        rem = _RPT % _C
        if rem:
            pltpu.sync_copy(
                ebufs[0].at[pl.ds(0, rem)],
                acc[k].at[pl.ds(r0 + _RPT - rem, rem)],
            )
    plsc.subcore_barrier()

    def _issue(b, ch):
        base = wid * _EPW + ch * _C
        pltpu.sync_copy(i0h.at[pl.ds(base, _C)], i0b[b])
        pltpu.sync_copy(i1h.at[pl.ds(base, _C)], i1b[b])
        pltpu.async_copy(xh.at[i0b[b]], comp[b][0], gsems[b])
        pltpu.async_copy(yh.at[i0b[b]], comp[b][1], gsems[b])
        pltpu.async_copy(zh.at[i0b[b]], comp[b][2], gsems[b])
        pltpu.async_copy(xh.at[i1b[b]], comp[b][3], gsems[b])
        pltpu.async_copy(yh.at[i1b[b]], comp[b][4], gsems[b])
        pltpu.async_copy(zh.at[i1b[b]], comp[b][5], gsems[b])

    def _drain_gathers(b):
        for j in range(6):
            pltpu.make_async_copy(xh.at[pl.ds(0, _C)], comp[b][j], gsems[b]).wait()

    def _phase(b, cur):
        _drain_gathers(b)
        nxt = cur + 1

        @pl.when(nxt < _NCHUNK)
        def _():
            _issue(1 - b, nxt)

        def _group(gi, _):
            sl = pl.ds(gi * 16, 16)
            dx = comp[b][0][sl] - comp[b][3][sl]
            dy = comp[b][1][sl] - comp[b][4][sl]
            dz = comp[b][2][sl] - comp[b][5][sl]
            d2 = dx * dx + dy * dy + dz * dz
            bits = lax.bitcast_convert_type(d2, jnp.int32)
            y = lax.bitcast_convert_type(
                jnp.int32(0x5F3759DF) - (bits >> 1), jnp.float32
            )
            for _i in range(3):
                y = y * (1.5 - 0.5 * d2 * y * y)
            dr = d2 * y
            for k in range(_NB):
                t = _SHIFTS[k] - dr
                ebufs[k][sl] = jnp.exp((-_BETA) * (t * t))
            return 0

        lax.fori_loop(0, _GPC, _group, 0)
        for k in range(_NB):
            pltpu.sync_copy(ebufs[k], acc[k].at[i1b[b]], add=True)

    _issue(0, 0)

    def _pair(i, _):
        _phase(0, 2 * i)
        _phase(1, 2 * i + 1)
        return 0

    lax.fori_loop(0, _NCHUNK // 2, _pair, 0)

    plsc.subcore_barrier()
    for k in range(_NB):
        pltpu.sync_copy(
            acc[k].at[pl.ds(r0, _RPT)], out.at[cid, k, pl.ds(r0, _RPT)]
        )


def _add_body(p_ref, o_ref):
    o_ref[...] = p_ref[0] + p_ref[1]


def kernel(R, neighbor_idx):
    R = R.astype(jnp.float32)
    x, y, z = R[:, 0], R[:, 1], R[:, 2]
    i0 = neighbor_idx[0].astype(jnp.int32)
    i1 = neighbor_idx[1].astype(jnp.int32)
    partial = _rbf_sc(x, y, z, i0, i1)
    p2 = partial.reshape(_NC, _NB * _NP // 128, 128)
    summed = pl.pallas_call(
        _add_body,
        out_shape=jax.ShapeDtypeStruct((_NB * _NP // 128, 128), jnp.float32),
    )(p2)
    return summed.reshape(_NB, _NP)[:, :_N].T
